# Initial kernel scaffold; baseline (speedup 1.0000x reference)
import functools

import jax
import jax.numpy as jnp
from jax.experimental import pallas as pl


_INTER = 512


def _moe_body(x_ref, gu_ref, dn_ref, idx_ref, wts_ref, out_ref):
    e = pl.program_id(0)
    x = x_ref[...]
    gu = gu_ref[0]
    proj = jax.lax.dot_general(
        x, gu, (((1,), (1,)), ((), ())),
        preferred_element_type=jnp.float32)
    gate = proj[:, :_INTER]
    up = proj[:, _INTER:]
    h = gate * jax.nn.sigmoid(gate) * up
    dn = dn_ref[0]
    out_e = jax.lax.dot_general(
        h, dn, (((1,), (1,)), ((), ())),
        preferred_element_type=jnp.float32)
    sel = (idx_ref[...] == e).astype(jnp.float32)
    w = jnp.sum(wts_ref[...] * sel, axis=1, keepdims=True)
    contrib = out_e * w

    @pl.when(e == 0)
    def _():
        out_ref[...] = contrib

    @pl.when(e != 0)
    def _():
        out_ref[...] += contrib


@jax.jit
def kernel(hidden_states, top_k_index, top_k_weights, gate_up_proj, down_proj):
    n, h = hidden_states.shape
    e = gate_up_proj.shape[0]
    i2 = gate_up_proj.shape[1]
    i = down_proj.shape[2]
    out = pl.pallas_call(
        _moe_body,
        grid=(e,),
        in_specs=[
            pl.BlockSpec((n, h), lambda ei: (0, 0)),
            pl.BlockSpec((1, i2, h), lambda ei: (ei, 0, 0)),
            pl.BlockSpec((1, h, i), lambda ei: (ei, 0, 0)),
            pl.BlockSpec(top_k_index.shape, lambda ei: (0, 0)),
            pl.BlockSpec(top_k_weights.shape, lambda ei: (0, 0)),
        ],
        out_specs=pl.BlockSpec((n, h), lambda ei: (0, 0)),
        out_shape=jax.ShapeDtypeStruct((n, h), jnp.float32),
    )(hidden_states, top_k_index.astype(jnp.int32), top_k_weights,
      gate_up_proj, down_proj)
    return out.astype(hidden_states.dtype)


# speedup vs baseline: 1.1100x; 1.1100x over previous
import functools

import jax
import jax.numpy as jnp
from jax.experimental import pallas as pl


_INTER = 512


def _moe_body(x_ref, gu_ref, dn_ref, idx_ref, wts_ref, out_ref):
    e = pl.program_id(0)
    x = x_ref[...]
    gu = gu_ref[0]
    proj = jax.lax.dot_general(
        x, gu, (((1,), (1,)), ((), ())),
        preferred_element_type=jnp.float32)
    gate = proj[:, :_INTER]
    up = proj[:, _INTER:]
    h = gate * jax.nn.sigmoid(gate) * up
    dn = dn_ref[0]
    out_e = jax.lax.dot_general(
        h, dn, (((1,), (1,)), ((), ())),
        preferred_element_type=jnp.float32)
    sel = (idx_ref[...] == e).astype(jnp.float32)
    w = jnp.sum(wts_ref[...] * sel, axis=1, keepdims=True)
    contrib = out_e * w

    @pl.when(e == 0)
    def _():
        out_ref[...] = contrib

    @pl.when(e != 0)
    def _():
        out_ref[...] += contrib


@jax.jit
def kernel(hidden_states, top_k_index, top_k_weights, gate_up_proj, down_proj):
    n, h = hidden_states.shape
    e = gate_up_proj.shape[0]
    i2 = gate_up_proj.shape[1]
    i = down_proj.shape[2]
    out = pl.pallas_call(
        _moe_body,
        grid=(e,),
        in_specs=[
            pl.BlockSpec((n, h), lambda ei: (0, 0)),
            pl.BlockSpec((1, i2, h), lambda ei: (ei, 0, 0)),
            pl.BlockSpec((1, h, i), lambda ei: (ei, 0, 0)),
            pl.BlockSpec(top_k_index.shape, lambda ei: (0, 0)),
            pl.BlockSpec(top_k_weights.shape, lambda ei: (0, 0)),
        ],
        out_specs=pl.BlockSpec((n, h), lambda ei: (0, 0)),
        out_shape=jax.ShapeDtypeStruct((n, h), jnp.float32),
    )(hidden_states, gate_up_proj, down_proj,
      top_k_index.astype(jnp.int32), top_k_weights)
    return out.astype(hidden_states.dtype)
